# baseline (speedup 1.0000x reference)
import functools

import jax
import jax.numpy as jnp
from jax import lax
from jax.experimental import pallas as pl
from jax.experimental.pallas import tpu as pltpu
from jax.experimental.pallas import tpu_sc as plsc

N = 10000
E = 320000
D_IN = 128
H = 8
C = 8
HID = 64

NC = 2
NS = 16
NW = NC * NS
LANES = 16

CH = 128
ETOT = E + N
CPW = 82
EPW = CPW * CH
EPAD = EPW * NW
DROW = 72
SLICES = (0, 16, 32, 48, 56)
NACC = 10240
ROWS_PER_TILE = NACC // NS
RCHUNK = 128

_EPS_DIV = 1e-16
_EPS_LN = 1e-5


def _expand_mat():
    col = lax.broadcasted_iota(jnp.int32, (H, DROW), 1)
    row = lax.broadcasted_iota(jnp.int32, (H, DROW), 0)
    head = jnp.where(col < HID, col // C, col - HID)
    return (head == row).astype(jnp.float32)


def _attn_tail(h, W, As_mat, Ad_mat, xe_ref, ase_ref, ade_ref, m80_ref):
    gb = h.shape[0]
    xh = jnp.dot(h, W, preferred_element_type=jnp.float32)
    asrc = jnp.dot(xh, As_mat, preferred_element_type=jnp.float32,
                   precision=jax.lax.Precision.HIGHEST)
    adst = jnp.dot(xh, Ad_mat, preferred_element_type=jnp.float32,
                   precision=jax.lax.Precision.HIGHEST)
    xe_ref[...] = jnp.concatenate(
        [xh, jnp.ones((gb, H), jnp.float32)], axis=1)
    expm = _expand_mat()
    ase_ref[...] = jnp.dot(asrc, expm, preferred_element_type=jnp.float32,
                           precision=jax.lax.Precision.HIGHEST)
    ade_ref[...] = jnp.dot(adst, expm, preferred_element_type=jnp.float32,
                           precision=jax.lax.Precision.HIGHEST)
    m8 = (jnp.max(asrc, axis=0, keepdims=True)
          + jnp.max(adst, axis=0, keepdims=True))
    m80 = jnp.dot(m8, expm, preferred_element_type=jnp.float32,
                  precision=jax.lax.Precision.HIGHEST)
    i = pl.program_id(0)

    @pl.when(i == 0)
    def _():
        m80_ref[...] = m80

    @pl.when(i > 0)
    def _():
        m80_ref[...] = jnp.maximum(m80_ref[...], m80)


def _tc_pre_body(x_ref, Win_ref, bin_ref, W_ref, Asm_ref, Adm_ref,
                 h_ref, xe_ref, ase_ref, ade_ref, m80_ref):
    h = jnp.maximum(
        jnp.dot(x_ref[...], Win_ref[...], preferred_element_type=jnp.float32)
        + bin_ref[...], 0.0)
    h_ref[...] = h
    _attn_tail(h, W_ref[...], Asm_ref[...], Adm_ref[...],
               xe_ref, ase_ref, ade_ref, m80_ref)


def _combine(p0, p1, hprev, bg, g, bb):
    acc = p0 + p1
    out_un = acc[:, :HID]
    s = acc[:, HID:HID + H]
    sinv = 1.0 / (s + _EPS_DIV)
    col = lax.broadcasted_iota(jnp.int32, (H, HID), 1) // C
    row = lax.broadcasted_iota(jnp.int32, (H, HID), 0)
    expm = (col == row).astype(jnp.float32)
    hn = out_un * jnp.dot(sinv, expm, preferred_element_type=jnp.float32, precision=jax.lax.Precision.HIGHEST) + bg
    mu = jnp.mean(hn, axis=-1, keepdims=True)
    var = jnp.mean((hn - mu) ** 2, axis=-1, keepdims=True)
    hn = (hn - mu) / jnp.sqrt(var + _EPS_LN) * g + bb
    return hprev + jnp.maximum(hn, 0.0)


def _tc_mid_body(p0_ref, p1_ref, hprev_ref, bg_ref, g_ref, bb_ref,
                 W_ref, Asm_ref, Adm_ref,
                 h_ref, xe_ref, ase_ref, ade_ref, m80_ref):
    h = _combine(p0_ref[...], p1_ref[...], hprev_ref[...],
                 bg_ref[...], g_ref[...], bb_ref[...])
    h_ref[...] = h
    _attn_tail(h, W_ref[...], Asm_ref[...], Adm_ref[...],
               xe_ref, ase_ref, ade_ref, m80_ref)


def _tc_final_body(p0_ref, p1_ref, hprev_ref, bg_ref, g_ref, bb_ref,
                   Wout_ref, bout_ref, y_ref):
    h = _combine(p0_ref[...], p1_ref[...], hprev_ref[...],
                 bg_ref[...], g_ref[...], bb_ref[...])
    y_ref[...] = (jnp.dot(h, Wout_ref[...], preferred_element_type=jnp.float32)
                  + bout_ref[...])


GB = 1000
NGB = N // GB


def _row(cols):
    return pl.BlockSpec((GB, cols), lambda i: (i, 0))


def _full(shape):
    return pl.BlockSpec(shape, lambda i: tuple(0 for _ in shape))


_NODE_OUTS = [
    jax.ShapeDtypeStruct((N, HID), jnp.float32),
    jax.ShapeDtypeStruct((N, DROW), jnp.float32),
    jax.ShapeDtypeStruct((N, DROW), jnp.float32),
    jax.ShapeDtypeStruct((N, DROW), jnp.float32),
    jax.ShapeDtypeStruct((1, DROW), jnp.float32),
]
_NODE_OUT_SPECS = [_row(HID), _row(DROW), _row(DROW), _row(DROW),
                   _full((1, DROW))]

_tc_pre = pl.pallas_call(
    _tc_pre_body, out_shape=_NODE_OUTS, grid=(NGB,),
    in_specs=[_row(D_IN), _full((D_IN, HID)), _full((1, HID)),
              _full((HID, HID)), _full((HID, H)), _full((HID, H))],
    out_specs=_NODE_OUT_SPECS)
_tc_mid = pl.pallas_call(
    _tc_mid_body, out_shape=_NODE_OUTS, grid=(NGB,),
    in_specs=[_row(DROW), _row(DROW), _row(HID), _full((1, HID)),
              _full((1, HID)), _full((1, HID)), _full((HID, HID)),
              _full((HID, H)), _full((HID, H))],
    out_specs=_NODE_OUT_SPECS)
_tc_final = pl.pallas_call(
    _tc_final_body, out_shape=jax.ShapeDtypeStruct((N, 1), jnp.float32),
    grid=(NGB,),
    in_specs=[_row(DROW), _row(DROW), _row(HID), _full((1, HID)),
              _full((1, HID)), _full((1, HID)), _full((HID, 1)),
              _full((1, 1))],
    out_specs=_row(1))


def _sc_edge_body(src_hbm, dst_hbm, ase_hbm, ade_hbm, big_hbm, m_hbm,
                  out_hbm,
                  srcv, dstv, sdst, r1, r2, xrows, wbuf, mv, accum,
                  gsem, ssem, isem):
    c = lax.axis_index("c")
    s = lax.axis_index("s")
    wid = s * NC + c

    def _zero_row(r, carry):
        for sl0 in SLICES:
            wbuf[0][r, pl.ds(sl0, LANES)] = jnp.zeros((LANES,), jnp.float32)
        return carry
    lax.fori_loop(0, CH, _zero_row, 0)
    row0 = s * ROWS_PER_TILE
    for k in range(ROWS_PER_TILE // RCHUNK):
        pltpu.sync_copy(wbuf[0].at[pl.ds(0, RCHUNK)],
                        accum.at[pl.ds(row0 + k * RCHUNK, RCHUNK)])
    plsc.subcore_barrier()

    pltpu.sync_copy(m_hbm, mv)
    mvs = [mv[pl.ds(sl0, LANES)] for sl0 in SLICES]

    base0 = wid * EPW

    def _issue_gathers(b):
        pltpu.async_copy(ase_hbm.at[srcv[b]], r1[b], gsem[b])
        pltpu.async_copy(ade_hbm.at[dstv[b]], r2[b], gsem[b])
        pltpu.async_copy(big_hbm.at[srcv[b]], xrows[b], gsem[b])

    def _wait_gathers(b):
        pltpu.make_async_copy(ase_hbm.at[srcv[b]], r1[b], gsem[b]).wait()
        pltpu.make_async_copy(ade_hbm.at[dstv[b]], r2[b], gsem[b]).wait()
        pltpu.make_async_copy(big_hbm.at[srcv[b]], xrows[b], gsem[b]).wait()

    def _issue_idx(cc, b):
        base = base0 + cc * CH
        pltpu.async_copy(src_hbm.at[pl.ds(base, CH)], srcv[b], isem[b])
        pltpu.async_copy(dst_hbm.at[pl.ds(base, CH)], dstv[b], isem[b])

    def _wait_idx(b):
        pltpu.make_async_copy(src_hbm.at[pl.ds(0, CH)], srcv[b],
                              isem[b]).wait()
        pltpu.make_async_copy(dst_hbm.at[pl.ds(0, CH)], dstv[b],
                              isem[b]).wait()

    def _compute(b):
        def _edge(e, carry2):
            for j, sl0 in enumerate(SLICES):
                sl = pl.ds(sl0, LANES)
                t = r1[b][e, sl] + r2[b][e, sl]
                ev = jnp.exp(jnp.maximum(t, 0.2 * t) - mvs[j])
                wbuf[b][e, sl] = xrows[b][e, sl] * ev
            return carry2
        lax.fori_loop(0, CH, _edge, 0)
        for j in range(CH // LANES):
            sl = pl.ds(LANES * j, LANES)
            sdst[b][sl] = dstv[b][sl]

    def _issue_scatter(b):
        pltpu.async_copy(wbuf[b], accum.at[sdst[b]], ssem[b], add=True)

    def _wait_scatter(b):
        pltpu.make_async_copy(wbuf[b], accum.at[sdst[b]], ssem[b]).wait()

    def _chunk_body(cc, b, first, issue_next_gather, issue_idx2):
        _wait_gathers(b)
        if issue_next_gather:
            _wait_idx(1 - b)
            _issue_gathers(1 - b)
        _compute(b)
        if not first:
            _wait_scatter(1 - b)
        _issue_scatter(b)
        if issue_idx2:
            _issue_idx(cc + 2, b)

    pltpu.sync_copy(src_hbm.at[pl.ds(base0, CH)], srcv[0])
    pltpu.sync_copy(dst_hbm.at[pl.ds(base0, CH)], dstv[0])
    _issue_gathers(0)
    _issue_idx(1, 1)

    _chunk_body(0, 0, first=True, issue_next_gather=True, issue_idx2=True)
    _chunk_body(1, 1, first=False, issue_next_gather=True, issue_idx2=True)

    def _pair(p, carry):
        c0 = 2 * p
        _chunk_body(c0, 0, first=False, issue_next_gather=True,
                    issue_idx2=True)
        _chunk_body(c0 + 1, 1, first=False, issue_next_gather=True,
                    issue_idx2=True)
        return carry
    lax.fori_loop(1, CPW // 2 - 1, _pair, 0)

    _chunk_body(CPW - 2, 0, first=False, issue_next_gather=True,
                issue_idx2=False)
    _chunk_body(CPW - 1, 1, first=False, issue_next_gather=False,
                issue_idx2=False)
    _wait_scatter(1)

    plsc.subcore_barrier()

    for k in range(ROWS_PER_TILE // RCHUNK):
        r0 = row0 + k * RCHUNK
        pltpu.sync_copy(accum.at[pl.ds(r0, RCHUNK)],
                        out_hbm.at[c, pl.ds(r0, RCHUNK)])


@functools.cache
def _make_sc_edge():
  return pl.kernel(
    _sc_edge_body,
    compiler_params=pltpu.CompilerParams(use_tc_tiling_on_sc=False),
    out_type=jax.ShapeDtypeStruct((NC, NACC, DROW), jnp.float32),
    mesh=plsc.VectorSubcoreMesh(core_axis_name="c", subcore_axis_name="s",
                                num_cores=NC, num_subcores=NS),
    scratch_types=[
        [pltpu.VMEM((CH,), jnp.int32)] * 2,
        [pltpu.VMEM((CH,), jnp.int32)] * 2,
        [pltpu.VMEM((CH,), jnp.int32)] * 2,
        [pltpu.VMEM((CH, DROW), jnp.float32)] * 2,
        [pltpu.VMEM((CH, DROW), jnp.float32)] * 2,
        [pltpu.VMEM((CH, DROW), jnp.float32)] * 2,
        [pltpu.VMEM((CH, DROW), jnp.float32)] * 2,
        pltpu.VMEM((DROW,), jnp.float32),
        pltpu.VMEM_SHARED((NACC, DROW), jnp.float32),
        [pltpu.SemaphoreType.DMA] * 2,
        [pltpu.SemaphoreType.DMA] * 2,
        [pltpu.SemaphoreType.DMA] * 2,
    ],
  )


def _expand_a(a):
    k = jnp.arange(HID)
    return jnp.zeros((HID, H), jnp.float32).at[k, k // C].set(a.reshape(HID))


def kernel(x, edge_index, W_in, b_in, W0, as0, ad0, bg0, g0, bb0,
           W1, as1, ad1, bg1, g1, bb1, W2, as2, ad2, bg2, g2, bb2,
           W_out, b_out):
    idt = edge_index.dtype
    loop = jnp.arange(N, dtype=idt)
    npad = EPAD - ETOT
    src = jnp.concatenate(
        [edge_index[0], loop, jnp.full((npad,), N, dtype=idt)])
    dst = jnp.concatenate(
        [edge_index[1], loop, jnp.zeros((npad,), dtype=idt)])
    src = src.astype(jnp.int32)
    dst = dst.astype(jnp.int32)

    layers = [(W0, as0, ad0, bg0, g0, bb0), (W1, as1, ad1, bg1, g1, bb1),
              (W2, as2, ad2, bg2, g2, bb2)]

    b_in2 = b_in.reshape(1, HID)
    W, a_s, a_d = layers[0][0], layers[0][1], layers[0][2]
    h, xe, ase, ade, m80 = _tc_pre(x, W_in, b_in2, W,
                                   _expand_a(a_s), _expand_a(a_d))

    for l in range(3):
        big = jnp.pad(xe, ((0, 1), (0, 0)))
        ase_p = jnp.pad(ase, ((0, 1), (0, 0)))
        parts = _make_sc_edge()(src, dst, ase_p, ade, big, m80.reshape(DROW))
        p0, p1 = parts[0], parts[1]
        bg, g, bb = layers[l][3], layers[l][4], layers[l][5]
        bg2, g2_, bb2 = (bg.reshape(1, HID), g.reshape(1, HID),
                         bb.reshape(1, HID))
        if l < 2:
            Wn, asn, adn = layers[l + 1][0], layers[l + 1][1], layers[l + 1][2]
            h, xe, ase, ade, m80 = _tc_mid(p0, p1, h, bg2, g2_, bb2, Wn,
                                           _expand_a(asn), _expand_a(adn))
        else:
            y = _tc_final(p0, p1, h, bg2, g2_, bb2, W_out,
                          b_out.reshape(1, 1))
    return y
